# Initial kernel scaffold; baseline (speedup 1.0000x reference)
import jax
import jax.numpy as jnp
from jax.experimental import pallas as pl


def kernel(x, edge_index, batch, W1, b1, W2, b2, W3, b3, LW1, Lb1, LW2, Lb2, LW3, Lb3):
    raise NotImplementedError("write your pallas kernel here")


# speedup vs baseline: 2.9484x; 2.9484x over previous
import functools

import jax
import jax.numpy as jnp
from jax import lax
from jax.experimental import pallas as pl
from jax.experimental.pallas import tpu as pltpu
from jax.experimental.pallas import tpu_sc as plsc

N = 10000
E = 160000
G = 16
NPAD = 10240
EPAD = 163840
EB = 128
EP_TILE = EPAD // 32
NB_BATCH = EP_TILE // EB
STRIPE = NPAD // 16
RB = 2048
NRB = NPAD // RB
F32 = jnp.float32
HI = jax.lax.Precision.HIGHEST


def _sc_mesh():
    return plsc.VectorSubcoreMesh(core_axis_name="c", subcore_axis_name="s")


def _make_sc_scatter(n_chunks):
    scratch = [
        pltpu.VMEM((EB,), jnp.int32),
        pltpu.VMEM((EB,), jnp.int32),
        pltpu.VMEM((EB, 128), F32),
        pltpu.VMEM_SHARED((NPAD, 128), F32),
        pltpu.SemaphoreType.DMA,
    ]
    out_t = [jax.ShapeDtypeStruct((2 * NPAD, 128), F32)] * n_chunks

    @functools.partial(pl.kernel, mesh=_sc_mesh(), out_type=out_t,
                       scratch_types=scratch)
    def k(src_hbm, dst_hbm, zero_hbm, *rest):
        p_refs = rest[:n_chunks]
        s_refs = rest[n_chunks:2 * n_chunks]
        srcv, dstv, rows, acc, sem = rest[2 * n_chunks:]
        core = lax.axis_index("c")
        sub = lax.axis_index("s")
        wid = sub * 2 + core
        e0 = wid * EP_TILE
        stripe0 = sub * STRIPE
        row0 = core * NPAD + stripe0
        for p_ref, s_ref in zip(p_refs, s_refs):
            pltpu.sync_copy(zero_hbm, acc.at[pl.ds(stripe0, STRIPE), :])
            plsc.subcore_barrier()

            def step(i, carry):
                off = e0 + i * EB
                pltpu.sync_copy(src_hbm.at[pl.ds(off, EB)], srcv)
                pltpu.sync_copy(dst_hbm.at[pl.ds(off, EB)], dstv)
                pltpu.async_copy(p_ref.at[srcv], rows, sem).wait()
                pltpu.sync_copy(rows, acc.at[dstv], add=True)
                return carry

            lax.fori_loop(0, NB_BATCH, step, 0)
            plsc.subcore_barrier()
            pltpu.sync_copy(acc.at[pl.ds(stripe0, STRIPE), :],
                            s_ref.at[pl.ds(row0, STRIPE), :])

    return k


def _t1_body(x_ref, w_ref, dega_ref, degb_ref, dinv_ref, *p_refs):
    c = pl.program_id(1)
    deg = dega_ref[:, 0:1] + degb_ref[:, 0:1] + 1.0
    dinv = jax.lax.rsqrt(deg)
    dinv_ref[...] = jnp.broadcast_to(dinv, (RB, 128))
    h = jnp.dot(x_ref[...], w_ref[...], preferred_element_type=F32,
                precision=HI)
    pch = dinv * h
    for j, pr in enumerate(p_refs):
        @pl.when(c == j)
        def _(pr=pr):
            pr[...] = pch


def _t1(xp, w1, degp):
    grid = (NRB, 4)
    return pl.pallas_call(
        _t1_body,
        grid=grid,
        in_specs=[
            pl.BlockSpec((RB, 256), lambda r, c: (r, 0)),
            pl.BlockSpec((256, 128), lambda r, c: (0, c)),
            pl.BlockSpec((RB, 128), lambda r, c: (r, 0)),
            pl.BlockSpec((RB, 128), lambda r, c: (r + NRB, 0)),
        ],
        out_specs=[pl.BlockSpec((RB, 128), lambda r, c: (r, 0))] * 5,
        out_shape=[jax.ShapeDtypeStruct((NPAD, 128), F32)] * 5,
    )(xp, w1, degp, degp)


def _make_t_mid(d_in, nc_out):
    nc_in = d_in // 128

    def body(*refs):
        sa = refs[:nc_in]
        sb = refs[nc_in:2 * nc_in]
        p = refs[2 * nc_in:3 * nc_in]
        dinv_ref, b_ref, w_ref = refs[3 * nc_in:3 * nc_in + 3]
        out_refs = refs[3 * nc_in + 3:3 * nc_in + 3 + nc_out]
        (u_ref,) = refs[3 * nc_in + 3 + nc_out:]
        c = pl.program_id(1)
        dinv = dinv_ref[:, 0:1]

        @pl.when(c == 0)
        def _():
            parts = [sa[j][...] + sb[j][...] + p[j][...] for j in range(nc_in)]
            s = jnp.concatenate(parts, axis=1)
            u_ref[...] = jnp.maximum(dinv * s + b_ref[...], 0.0)

        h = jnp.dot(u_ref[...], w_ref[...], preferred_element_type=F32,
                    precision=HI)
        pch = dinv * h
        for j, pr in enumerate(out_refs):
            @pl.when(c == j)
            def _(pr=pr):
                pr[...] = pch

    def call(s_parts, p_parts, dinv, b_row, w):
        grid = (NRB, nc_out)
        in_specs = (
            [pl.BlockSpec((RB, 128), lambda r, c: (r, 0))] * nc_in
            + [pl.BlockSpec((RB, 128), lambda r, c: (r + NRB, 0))] * nc_in
            + [pl.BlockSpec((RB, 128), lambda r, c: (r, 0))] * nc_in
            + [
                pl.BlockSpec((RB, 128), lambda r, c: (r, 0)),
                pl.BlockSpec((1, d_in), lambda r, c: (0, 0)),
                pl.BlockSpec((d_in, 128), lambda r, c: (0, c)),
            ]
        )
        return pl.pallas_call(
            body,
            grid=grid,
            in_specs=in_specs,
            out_specs=[pl.BlockSpec((RB, 128), lambda r, c: (r, 0))] * nc_out,
            out_shape=[jax.ShapeDtypeStruct((NPAD, 128), F32)] * nc_out,
            scratch_shapes=[pltpu.VMEM((RB, d_in), F32)],
        )(*s_parts, *s_parts, *p_parts, dinv, b_row, w)

    return call


def _t4_body(s0_ref, s1_ref, p0_ref, p1_ref, dinv_ref, b_ref, bat_ref,
             lw1_ref, lb1_ref, lw2_ref, lb2_ref, lw3_ref, lb3_ref, out_ref):
    dinv = dinv_ref[:, 0:1]
    u0 = s0_ref[0:NPAD, :] + s0_ref[NPAD:2 * NPAD, :] + p0_ref[...]
    u1 = s1_ref[0:NPAD, :] + s1_ref[NPAD:2 * NPAD, :] + p1_ref[...]
    u = jnp.concatenate([u0, u1], axis=1)
    u = jnp.maximum(dinv * u + b_ref[...], 0.0)
    bat = bat_ref[...]
    onehot = (bat == lax.broadcasted_iota(jnp.int32, (G, NPAD), 0)).astype(F32)
    gsum = jnp.dot(onehot, u, preferred_element_type=F32, precision=HI)
    counts = jnp.sum(onehot, axis=1, keepdims=True)
    g = gsum / jnp.maximum(counts, 1.0)
    g = jnp.maximum(jnp.dot(g, lw1_ref[...], preferred_element_type=F32,
                            precision=HI) + lb1_ref[...], 0.0)
    g = jnp.maximum(jnp.dot(g, lw2_ref[...], preferred_element_type=F32,
                            precision=HI) + lb2_ref[...], 0.0)
    g = jnp.maximum(jnp.dot(g, lw3_ref[...], preferred_element_type=F32,
                            precision=HI) + lb3_ref[...], 0.0)
    out_ref[...] = g


def _t4(s_parts, p_parts, dinv, b_row, bat_row, lw1, lb1, lw2, lb2, lw3, lb3):
    return pl.pallas_call(
        _t4_body,
        out_shape=jax.ShapeDtypeStruct((G, 8), F32),
    )(*s_parts, *p_parts, dinv, b_row, bat_row,
      lw1, lb1, lw2, lb2, lw3, lb3)


_sc_scatter = functools.lru_cache(maxsize=None)(_make_sc_scatter)
_t2 = _make_t_mid(512, 4)
_t3 = _make_t_mid(512, 2)


def kernel(x, edge_index, batch, W1, b1, W2, b2, W3, b3,
           LW1, Lb1, LW2, Lb2, LW3, Lb3):
    src = jnp.concatenate(
        [edge_index[0].astype(jnp.int32),
         jnp.zeros((EPAD - E,), jnp.int32)])
    dst = jnp.concatenate(
        [edge_index[1].astype(jnp.int32),
         jnp.full((EPAD - E,), N, jnp.int32)])
    bat_row = jnp.concatenate(
        [batch.astype(jnp.int32),
         jnp.full((NPAD - N,), G, jnp.int32)]).reshape(1, NPAD)
    xp = jnp.pad(x, ((0, NPAD - N), (0, 0)))
    zeros128 = jnp.zeros((STRIPE, 128), F32)
    ones128 = jnp.ones((NPAD, 128), F32)

    degp = _sc_scatter(1)(dst, dst, zeros128, ones128)
    if isinstance(degp, (list, tuple)):
        degp = degp[0]
    dinv, *p1 = _t1(xp, W1, degp)
    s1 = _sc_scatter(4)(src, dst, zeros128, *p1)
    p2 = _t2(s1, p1, dinv, b1.reshape(1, -1), W2)
    s2 = _sc_scatter(4)(src, dst, zeros128, *p2)
    p3 = _t3(s2, p2, dinv, b2.reshape(1, -1), W3)
    s3 = _sc_scatter(2)(src, dst, zeros128, *p3)
    return _t4(s3, p3, dinv, b3.reshape(1, -1), bat_row,
               LW1, Lb1.reshape(1, -1), LW2, Lb2.reshape(1, -1),
               LW3, Lb3.reshape(1, -1))


# speedup vs baseline: 3.8770x; 1.3150x over previous
import functools

import jax
import jax.numpy as jnp
from jax import lax
from jax.experimental import pallas as pl
from jax.experimental.pallas import tpu as pltpu
from jax.experimental.pallas import tpu_sc as plsc

N = 10000
E = 160000
G = 16
NPAD = 10240
EPAD = 163840
EB = 128
EP_TILE = EPAD // 32
NB_BATCH = EP_TILE // EB
STRIPE = NPAD // 16
RB = 2048
NRB = NPAD // RB
F32 = jnp.float32
HI = jax.lax.Precision.HIGHEST


def _sc_mesh():
    return plsc.VectorSubcoreMesh(core_axis_name="c", subcore_axis_name="s")


def _make_sc_scatter(n_chunks):
    scratch = [
        pltpu.VMEM((NB_BATCH, EB), jnp.int32),
        pltpu.VMEM((NB_BATCH, EB), jnp.int32),
        pltpu.VMEM((2, EB, 128), F32),
        pltpu.VMEM_SHARED((NPAD, 128), F32),
        pltpu.SemaphoreType.DMA((2,)),
    ]
    out_t = [jax.ShapeDtypeStruct((2 * NPAD, 128), F32)] * n_chunks

    @functools.partial(pl.kernel, mesh=_sc_mesh(), out_type=out_t,
                       scratch_types=scratch)
    def k(src_hbm, dst_hbm, zero_hbm, *rest):
        p_refs = rest[:n_chunks]
        s_refs = rest[n_chunks:2 * n_chunks]
        src_all, dst_all, rows, acc, sem = rest[2 * n_chunks:]
        core = lax.axis_index("c")
        sub = lax.axis_index("s")
        wid = sub * 2 + core
        b0 = wid * NB_BATCH
        stripe0 = sub * STRIPE
        row0 = core * NPAD + stripe0
        pltpu.sync_copy(src_hbm.at[pl.ds(b0, NB_BATCH), :], src_all)
        pltpu.sync_copy(dst_hbm.at[pl.ds(b0, NB_BATCH), :], dst_all)
        for p_ref, s_ref in zip(p_refs, s_refs):
            pltpu.sync_copy(zero_hbm, acc.at[pl.ds(stripe0, STRIPE), :])
            plsc.subcore_barrier()

            pltpu.async_copy(p_ref.at[src_all.at[0]], rows.at[0], sem.at[0])

            def step(i, carry):
                par = lax.rem(i, 2)
                nxt = lax.rem(i + 1, 2)

                @pl.when(i + 1 < NB_BATCH)
                def _():
                    pltpu.async_copy(p_ref.at[src_all.at[i + 1]],
                                     rows.at[nxt], sem.at[nxt])

                pltpu.make_async_copy(p_ref.at[src_all.at[i]],
                                      rows.at[par], sem.at[par]).wait()
                pltpu.sync_copy(rows.at[par], acc.at[dst_all.at[i]], add=True)
                return carry

            lax.fori_loop(0, NB_BATCH, step, 0)
            plsc.subcore_barrier()
            pltpu.sync_copy(acc.at[pl.ds(stripe0, STRIPE), :],
                            s_ref.at[pl.ds(row0, STRIPE), :])

    return k


def _t1_body(x_ref, w_ref, dega_ref, degb_ref, dinv_ref, *p_refs):
    c = pl.program_id(1)
    deg = dega_ref[:, 0:1] + degb_ref[:, 0:1] + 1.0
    dinv = jax.lax.rsqrt(deg)
    dinv_ref[...] = jnp.broadcast_to(dinv, (RB, 128))
    h = jnp.dot(x_ref[...], w_ref[...], preferred_element_type=F32,
                precision=HI)
    pch = dinv * h
    for j, pr in enumerate(p_refs):
        @pl.when(c == j)
        def _(pr=pr):
            pr[...] = pch


def _t1(xp, w1, degp):
    grid = (NRB, 4)
    return pl.pallas_call(
        _t1_body,
        grid=grid,
        in_specs=[
            pl.BlockSpec((RB, 256), lambda r, c: (r, 0)),
            pl.BlockSpec((256, 128), lambda r, c: (0, c)),
            pl.BlockSpec((RB, 128), lambda r, c: (r, 0)),
            pl.BlockSpec((RB, 128), lambda r, c: (r + NRB, 0)),
        ],
        out_specs=[pl.BlockSpec((RB, 128), lambda r, c: (r, 0))] * 5,
        out_shape=[jax.ShapeDtypeStruct((NPAD, 128), F32)] * 5,
    )(xp, w1, degp, degp)


def _make_t_mid(d_in, nc_out):
    nc_in = d_in // 128

    def body(*refs):
        sa = refs[:nc_in]
        sb = refs[nc_in:2 * nc_in]
        p = refs[2 * nc_in:3 * nc_in]
        dinv_ref, b_ref, w_ref = refs[3 * nc_in:3 * nc_in + 3]
        out_refs = refs[3 * nc_in + 3:3 * nc_in + 3 + nc_out]
        (u_ref,) = refs[3 * nc_in + 3 + nc_out:]
        c = pl.program_id(1)
        dinv = dinv_ref[:, 0:1]

        @pl.when(c == 0)
        def _():
            parts = [sa[j][...] + sb[j][...] + p[j][...] for j in range(nc_in)]
            s = jnp.concatenate(parts, axis=1)
            u_ref[...] = jnp.maximum(dinv * s + b_ref[...], 0.0)

        h = jnp.dot(u_ref[...], w_ref[...], preferred_element_type=F32,
                    precision=HI)
        pch = dinv * h
        for j, pr in enumerate(out_refs):
            @pl.when(c == j)
            def _(pr=pr):
                pr[...] = pch

    def call(s_parts, p_parts, dinv, b_row, w):
        grid = (NRB, nc_out)
        in_specs = (
            [pl.BlockSpec((RB, 128), lambda r, c: (r, 0))] * nc_in
            + [pl.BlockSpec((RB, 128), lambda r, c: (r + NRB, 0))] * nc_in
            + [pl.BlockSpec((RB, 128), lambda r, c: (r, 0))] * nc_in
            + [
                pl.BlockSpec((RB, 128), lambda r, c: (r, 0)),
                pl.BlockSpec((1, d_in), lambda r, c: (0, 0)),
                pl.BlockSpec((d_in, 128), lambda r, c: (0, c)),
            ]
        )
        return pl.pallas_call(
            body,
            grid=grid,
            in_specs=in_specs,
            out_specs=[pl.BlockSpec((RB, 128), lambda r, c: (r, 0))] * nc_out,
            out_shape=[jax.ShapeDtypeStruct((NPAD, 128), F32)] * nc_out,
            scratch_shapes=[pltpu.VMEM((RB, d_in), F32)],
        )(*s_parts, *s_parts, *p_parts, dinv, b_row, w)

    return call


def _t4_body(s0_ref, s1_ref, p0_ref, p1_ref, dinv_ref, b_ref, bat_ref,
             lw1_ref, lb1_ref, lw2_ref, lb2_ref, lw3_ref, lb3_ref, out_ref):
    dinv = dinv_ref[:, 0:1]
    u0 = s0_ref[0:NPAD, :] + s0_ref[NPAD:2 * NPAD, :] + p0_ref[...]
    u1 = s1_ref[0:NPAD, :] + s1_ref[NPAD:2 * NPAD, :] + p1_ref[...]
    u = jnp.concatenate([u0, u1], axis=1)
    u = jnp.maximum(dinv * u + b_ref[...], 0.0)
    bat = bat_ref[...]
    onehot = (bat == lax.broadcasted_iota(jnp.int32, (G, NPAD), 0)).astype(F32)
    gsum = jnp.dot(onehot, u, preferred_element_type=F32, precision=HI)
    counts = jnp.sum(onehot, axis=1, keepdims=True)
    g = gsum / jnp.maximum(counts, 1.0)
    g = jnp.maximum(jnp.dot(g, lw1_ref[...], preferred_element_type=F32,
                            precision=HI) + lb1_ref[...], 0.0)
    g = jnp.maximum(jnp.dot(g, lw2_ref[...], preferred_element_type=F32,
                            precision=HI) + lb2_ref[...], 0.0)
    g = jnp.maximum(jnp.dot(g, lw3_ref[...], preferred_element_type=F32,
                            precision=HI) + lb3_ref[...], 0.0)
    out_ref[...] = g


def _t4(s_parts, p_parts, dinv, b_row, bat_row, lw1, lb1, lw2, lb2, lw3, lb3):
    return pl.pallas_call(
        _t4_body,
        out_shape=jax.ShapeDtypeStruct((G, 8), F32),
    )(*s_parts, *p_parts, dinv, b_row, bat_row,
      lw1, lb1, lw2, lb2, lw3, lb3)


_sc_scatter = functools.lru_cache(maxsize=None)(_make_sc_scatter)
_t2 = _make_t_mid(512, 4)
_t3 = _make_t_mid(512, 2)


def kernel(x, edge_index, batch, W1, b1, W2, b2, W3, b3,
           LW1, Lb1, LW2, Lb2, LW3, Lb3):
    src = jnp.concatenate(
        [edge_index[0].astype(jnp.int32),
         jnp.zeros((EPAD - E,), jnp.int32)]).reshape(EPAD // EB, EB)
    dst = jnp.concatenate(
        [edge_index[1].astype(jnp.int32),
         jnp.full((EPAD - E,), N, jnp.int32)]).reshape(EPAD // EB, EB)
    bat_row = jnp.concatenate(
        [batch.astype(jnp.int32),
         jnp.full((NPAD - N,), G, jnp.int32)]).reshape(1, NPAD)
    xp = jnp.pad(x, ((0, NPAD - N), (0, 0)))
    zeros128 = jnp.zeros((STRIPE, 128), F32)
    ones128 = jnp.ones((NPAD, 128), F32)

    degp = _sc_scatter(1)(dst, dst, zeros128, ones128)
    if isinstance(degp, (list, tuple)):
        degp = degp[0]
    dinv, *p1 = _t1(xp, W1, degp)
    s1 = _sc_scatter(4)(src, dst, zeros128, *p1)
    p2 = _t2(s1, p1, dinv, b1.reshape(1, -1), W2)
    s2 = _sc_scatter(4)(src, dst, zeros128, *p2)
    p3 = _t3(s2, p2, dinv, b2.reshape(1, -1), W3)
    s3 = _sc_scatter(2)(src, dst, zeros128, *p3)
    return _t4(s3, p3, dinv, b3.reshape(1, -1), bat_row,
               LW1, Lb1.reshape(1, -1), LW2, Lb2.reshape(1, -1),
               LW3, Lb3.reshape(1, -1))


# speedup vs baseline: 5.2818x; 1.3624x over previous
import functools

import jax
import jax.numpy as jnp
from jax import lax
from jax.experimental import pallas as pl
from jax.experimental.pallas import tpu as pltpu
from jax.experimental.pallas import tpu_sc as plsc

N = 10000
E = 160000
G = 16
NPAD = 10240
EPAD = 163840
EB = 128
NB2 = EPAD // 16 // EB
NBH = NB2 // 2
STRIPE = NPAD // 16
RB = 2048
NRB = NPAD // RB
F32 = jnp.float32
HI = jax.lax.Precision.HIGHEST


def _sc_mesh():
    return plsc.VectorSubcoreMesh(core_axis_name="c", subcore_axis_name="s")


def _make_sc_scatter(n_chunks):
    cpc = (n_chunks + 1) // 2
    scratch = [
        pltpu.VMEM((NBH, EB), jnp.int32),
        pltpu.VMEM((NBH, EB), jnp.int32),
        pltpu.VMEM((2, EB, 128), F32),
        pltpu.VMEM_SHARED((NPAD, 128), F32),
        pltpu.SemaphoreType.DMA((2,)),
    ]
    out_t = [jax.ShapeDtypeStruct((NPAD, 128), F32)] * n_chunks

    @functools.partial(pl.kernel, mesh=_sc_mesh(), out_type=out_t,
                       scratch_types=scratch)
    def k(src_hbm, dst_hbm, zero_hbm, *rest):
        p_refs = rest[:n_chunks]
        s_refs = rest[n_chunks:2 * n_chunks]
        src_all, dst_all, rows, acc, sem = rest[2 * n_chunks:]
        core = lax.axis_index("c")
        sub = lax.axis_index("s")
        b0 = sub * NB2
        stripe0 = sub * STRIPE

        def process(p_ref, s_ref):
            pltpu.sync_copy(zero_hbm, acc.at[pl.ds(stripe0, STRIPE), :])
            plsc.subcore_barrier()

            for h in range(2):
                pltpu.sync_copy(src_hbm.at[pl.ds(b0 + h * NBH, NBH), :],
                                src_all)
                pltpu.sync_copy(dst_hbm.at[pl.ds(b0 + h * NBH, NBH), :],
                                dst_all)

                pltpu.async_copy(p_ref.at[src_all.at[0]], rows.at[0],
                                 sem.at[0])

                def step(i, carry):
                    par = lax.rem(i, 2)
                    nxt = lax.rem(i + 1, 2)

                    @pl.when(i + 1 < NBH)
                    def _():
                        pltpu.async_copy(p_ref.at[src_all.at[i + 1]],
                                         rows.at[nxt], sem.at[nxt])

                    pltpu.make_async_copy(p_ref.at[src_all.at[i]],
                                          rows.at[par], sem.at[par]).wait()
                    pltpu.sync_copy(rows.at[par], acc.at[dst_all.at[i]],
                                    add=True)
                    return carry

                lax.fori_loop(0, NBH, step, 0)
            plsc.subcore_barrier()
            pltpu.sync_copy(acc.at[pl.ds(stripe0, STRIPE), :],
                            s_ref.at[pl.ds(stripe0, STRIPE), :])

        for li in range(cpc):
            c0 = li
            c1 = cpc + li

            @pl.when(core == 0)
            def _(c0=c0):
                process(p_refs[c0], s_refs[c0])

            if c1 < n_chunks:
                @pl.when(core == 1)
                def _(c1=c1):
                    process(p_refs[c1], s_refs[c1])

    return k


def _t1_body(x_ref, w_ref, deg_ref, dinv_ref, *p_refs):
    c = pl.program_id(1)
    deg = deg_ref[:, 0:1] + 1.0
    dinv = jax.lax.rsqrt(deg)
    dinv_ref[...] = jnp.broadcast_to(dinv, (RB, 128))
    h = jnp.dot(x_ref[...], w_ref[...], preferred_element_type=F32,
                precision=HI)
    pch = dinv * h
    for j, pr in enumerate(p_refs):
        @pl.when(c == j)
        def _(pr=pr):
            pr[...] = pch


def _t1(xp, w1, degp):
    grid = (NRB, 4)
    return pl.pallas_call(
        _t1_body,
        grid=grid,
        in_specs=[
            pl.BlockSpec((RB, 256), lambda r, c: (r, 0)),
            pl.BlockSpec((256, 128), lambda r, c: (0, c)),
            pl.BlockSpec((RB, 128), lambda r, c: (r, 0)),
        ],
        out_specs=[pl.BlockSpec((RB, 128), lambda r, c: (r, 0))] * 5,
        out_shape=[jax.ShapeDtypeStruct((NPAD, 128), F32)] * 5,
    )(xp, w1, degp)


def _make_t_mid(d_in, nc_out):
    nc_in = d_in // 128

    def body(*refs):
        s = refs[:nc_in]
        p = refs[nc_in:2 * nc_in]
        dinv_ref, b_ref, w_ref = refs[2 * nc_in:2 * nc_in + 3]
        out_refs = refs[2 * nc_in + 3:2 * nc_in + 3 + nc_out]
        (u_ref,) = refs[2 * nc_in + 3 + nc_out:]
        c = pl.program_id(1)
        dinv = dinv_ref[:, 0:1]

        @pl.when(c == 0)
        def _():
            parts = [s[j][...] + p[j][...] for j in range(nc_in)]
            sp = jnp.concatenate(parts, axis=1)
            u_ref[...] = jnp.maximum(dinv * sp + b_ref[...], 0.0)

        h = jnp.dot(u_ref[...], w_ref[...], preferred_element_type=F32,
                    precision=HI)
        pch = dinv * h
        for j, pr in enumerate(out_refs):
            @pl.when(c == j)
            def _(pr=pr):
                pr[...] = pch

    def call(s_parts, p_parts, dinv, b_row, w):
        grid = (NRB, nc_out)
        in_specs = (
            [pl.BlockSpec((RB, 128), lambda r, c: (r, 0))] * nc_in
            + [pl.BlockSpec((RB, 128), lambda r, c: (r, 0))] * nc_in
            + [
                pl.BlockSpec((RB, 128), lambda r, c: (r, 0)),
                pl.BlockSpec((1, d_in), lambda r, c: (0, 0)),
                pl.BlockSpec((d_in, 128), lambda r, c: (0, c)),
            ]
        )
        return pl.pallas_call(
            body,
            grid=grid,
            in_specs=in_specs,
            out_specs=[pl.BlockSpec((RB, 128), lambda r, c: (r, 0))] * nc_out,
            out_shape=[jax.ShapeDtypeStruct((NPAD, 128), F32)] * nc_out,
            scratch_shapes=[pltpu.VMEM((RB, d_in), F32)],
        )(*s_parts, *p_parts, dinv, b_row, w)

    return call


def _t4_body(s0_ref, s1_ref, p0_ref, p1_ref, dinv_ref, b_ref, bat_ref,
             lw1_ref, lb1_ref, lw2_ref, lb2_ref, lw3_ref, lb3_ref, out_ref):
    dinv = dinv_ref[:, 0:1]
    u0 = s0_ref[...] + p0_ref[...]
    u1 = s1_ref[...] + p1_ref[...]
    u = jnp.concatenate([u0, u1], axis=1)
    u = jnp.maximum(dinv * u + b_ref[...], 0.0)
    bat = bat_ref[...]
    onehot = (bat == lax.broadcasted_iota(jnp.int32, (G, NPAD), 0)).astype(F32)
    gsum = jnp.dot(onehot, u, preferred_element_type=F32, precision=HI)
    counts = jnp.sum(onehot, axis=1, keepdims=True)
    g = gsum / jnp.maximum(counts, 1.0)
    g = jnp.maximum(jnp.dot(g, lw1_ref[...], preferred_element_type=F32,
                            precision=HI) + lb1_ref[...], 0.0)
    g = jnp.maximum(jnp.dot(g, lw2_ref[...], preferred_element_type=F32,
                            precision=HI) + lb2_ref[...], 0.0)
    g = jnp.maximum(jnp.dot(g, lw3_ref[...], preferred_element_type=F32,
                            precision=HI) + lb3_ref[...], 0.0)
    out_ref[...] = g


def _t4(s_parts, p_parts, dinv, b_row, bat_row, lw1, lb1, lw2, lb2, lw3, lb3):
    return pl.pallas_call(
        _t4_body,
        out_shape=jax.ShapeDtypeStruct((G, 8), F32),
    )(*s_parts, *p_parts, dinv, b_row, bat_row,
      lw1, lb1, lw2, lb2, lw3, lb3)


_sc_scatter = functools.lru_cache(maxsize=None)(_make_sc_scatter)
_t2 = _make_t_mid(512, 4)
_t3 = _make_t_mid(512, 2)


def kernel(x, edge_index, batch, W1, b1, W2, b2, W3, b3,
           LW1, Lb1, LW2, Lb2, LW3, Lb3):
    src = jnp.concatenate(
        [edge_index[0].astype(jnp.int32),
         jnp.zeros((EPAD - E,), jnp.int32)]).reshape(EPAD // EB, EB)
    dst = jnp.concatenate(
        [edge_index[1].astype(jnp.int32),
         jnp.full((EPAD - E,), N, jnp.int32)]).reshape(EPAD // EB, EB)
    bat_row = jnp.concatenate(
        [batch.astype(jnp.int32),
         jnp.full((NPAD - N,), G, jnp.int32)]).reshape(1, NPAD)
    xp = jnp.pad(x, ((0, NPAD - N), (0, 0)))
    zeros128 = jnp.zeros((STRIPE, 128), F32)
    ones128 = jnp.ones((NPAD, 128), F32)

    degp = _sc_scatter(1)(dst, dst, zeros128, ones128)
    if isinstance(degp, (list, tuple)):
        degp = degp[0]
    dinv, *p1 = _t1(xp, W1, degp)
    s1 = _sc_scatter(4)(src, dst, zeros128, *p1)
    p2 = _t2(s1, p1, dinv, b1.reshape(1, -1), W2)
    s2 = _sc_scatter(4)(src, dst, zeros128, *p2)
    p3 = _t3(s2, p2, dinv, b2.reshape(1, -1), W3)
    s3 = _sc_scatter(2)(src, dst, zeros128, *p3)
    return _t4(s3, p3, dinv, b3.reshape(1, -1), bat_row,
               LW1, Lb1.reshape(1, -1), LW2, Lb2.reshape(1, -1),
               LW3, Lb3.reshape(1, -1))


# speedup vs baseline: 5.9191x; 1.1207x over previous
import functools

import jax
import jax.numpy as jnp
from jax import lax
from jax.experimental import pallas as pl
from jax.experimental.pallas import tpu as pltpu
from jax.experimental.pallas import tpu_sc as plsc

N = 10000
E = 160000
G = 16
NPAD = 10240
EPAD = 163840
EB = 128
NB2 = EPAD // 16 // EB
NBH = NB2 // 2
STRIPE = NPAD // 16
RB = 2048
NRB = NPAD // RB
F32 = jnp.float32
HI = jax.lax.Precision.HIGHEST


def _sc_mesh():
    return plsc.VectorSubcoreMesh(core_axis_name="c", subcore_axis_name="s")


def _make_sc_deg():
    scratch = [
        pltpu.VMEM((NBH, EB), jnp.int32),
        pltpu.VMEM((EB, 128), F32),
        pltpu.VMEM_SHARED((NPAD, 128), F32),
    ]
    out_t = [jax.ShapeDtypeStruct((NPAD, 128), F32)] * 2

    @functools.partial(pl.kernel, mesh=_sc_mesh(), out_type=out_t,
                       scratch_types=scratch)
    def k(dst_hbm, ones_hbm, zero_hbm, d0_hbm, d1_hbm, dst_all, ones_v, acc):
        core = lax.axis_index("c")
        sub = lax.axis_index("s")
        wid = sub * 2 + core
        b0 = wid * NBH
        stripe0 = sub * STRIPE
        pltpu.sync_copy(ones_hbm, ones_v)
        pltpu.sync_copy(dst_hbm.at[pl.ds(b0, NBH), :], dst_all)
        pltpu.sync_copy(zero_hbm, acc.at[pl.ds(stripe0, STRIPE), :])
        plsc.subcore_barrier()

        def step(i, carry):
            pltpu.sync_copy(ones_v, acc.at[dst_all.at[i]], add=True)
            return carry

        lax.fori_loop(0, NBH, step, 0)
        plsc.subcore_barrier()

        @pl.when(core == 0)
        def _():
            pltpu.sync_copy(acc.at[pl.ds(stripe0, STRIPE), :],
                            d0_hbm.at[pl.ds(stripe0, STRIPE), :])

        @pl.when(core == 1)
        def _():
            pltpu.sync_copy(acc.at[pl.ds(stripe0, STRIPE), :],
                            d1_hbm.at[pl.ds(stripe0, STRIPE), :])

    return k


def _make_sc_scatter(n_chunks):
    cpc = (n_chunks + 1) // 2
    scratch = [
        pltpu.VMEM((NBH, EB), jnp.int32),
        pltpu.VMEM((NBH, EB), jnp.int32),
        pltpu.VMEM((2, EB, 128), F32),
        pltpu.VMEM_SHARED((NPAD, 128), F32),
        pltpu.SemaphoreType.DMA((2,)),
    ]
    out_t = [jax.ShapeDtypeStruct((NPAD, 128), F32)] * n_chunks

    @functools.partial(pl.kernel, mesh=_sc_mesh(), out_type=out_t,
                       scratch_types=scratch)
    def k(src_hbm, dst_hbm, zero_hbm, *rest):
        p_refs = rest[:n_chunks]
        s_refs = rest[n_chunks:2 * n_chunks]
        src_all, dst_all, rows, acc, sem = rest[2 * n_chunks:]
        core = lax.axis_index("c")
        sub = lax.axis_index("s")
        b0 = sub * NB2
        stripe0 = sub * STRIPE

        def process(p_ref, s_ref):
            pltpu.sync_copy(zero_hbm, acc.at[pl.ds(stripe0, STRIPE), :])
            plsc.subcore_barrier()

            for h in range(2):
                pltpu.sync_copy(src_hbm.at[pl.ds(b0 + h * NBH, NBH), :],
                                src_all)
                pltpu.sync_copy(dst_hbm.at[pl.ds(b0 + h * NBH, NBH), :],
                                dst_all)

                pltpu.async_copy(p_ref.at[src_all.at[0]], rows.at[0],
                                 sem.at[0])

                def step(i, carry):
                    par = lax.rem(i, 2)
                    nxt = lax.rem(i + 1, 2)

                    @pl.when(i + 1 < NBH)
                    def _():
                        pltpu.async_copy(p_ref.at[src_all.at[i + 1]],
                                         rows.at[nxt], sem.at[nxt])

                    pltpu.make_async_copy(p_ref.at[src_all.at[i]],
                                          rows.at[par], sem.at[par]).wait()
                    pltpu.sync_copy(rows.at[par], acc.at[dst_all.at[i]],
                                    add=True)
                    return carry

                lax.fori_loop(0, NBH, step, 0)
            plsc.subcore_barrier()
            pltpu.sync_copy(acc.at[pl.ds(stripe0, STRIPE), :],
                            s_ref.at[pl.ds(stripe0, STRIPE), :])

        for li in range(cpc):
            c0 = li
            c1 = cpc + li

            @pl.when(core == 0)
            def _(c0=c0):
                process(p_refs[c0], s_refs[c0])

            if c1 < n_chunks:
                @pl.when(core == 1)
                def _(c1=c1):
                    process(p_refs[c1], s_refs[c1])

    return k


def _t1_body(x_ref, w_ref, dega_ref, degb_ref, dinv_ref, *p_refs):
    c = pl.program_id(1)
    deg = dega_ref[:, 0:1] + degb_ref[:, 0:1] + 1.0
    dinv = jax.lax.rsqrt(deg)
    dinv_ref[...] = jnp.broadcast_to(dinv, (RB, 128))
    h = jnp.dot(x_ref[...], w_ref[...], preferred_element_type=F32,
                precision=HI)
    pch = dinv * h
    for j, pr in enumerate(p_refs):
        @pl.when(c == j)
        def _(pr=pr):
            pr[...] = pch


def _t1(xp, w1, dega, degb):
    grid = (NRB, 4)
    return pl.pallas_call(
        _t1_body,
        grid=grid,
        in_specs=[
            pl.BlockSpec((RB, 256), lambda r, c: (r, 0)),
            pl.BlockSpec((256, 128), lambda r, c: (0, c)),
            pl.BlockSpec((RB, 128), lambda r, c: (r, 0)),
            pl.BlockSpec((RB, 128), lambda r, c: (r, 0)),
        ],
        out_specs=[pl.BlockSpec((RB, 128), lambda r, c: (r, 0))] * 5,
        out_shape=[jax.ShapeDtypeStruct((NPAD, 128), F32)] * 5,
    )(xp, w1, dega, degb)


def _make_t_mid(d_in, nc_out):
    nc_in = d_in // 128

    def body(*refs):
        s = refs[:nc_in]
        p = refs[nc_in:2 * nc_in]
        dinv_ref, b_ref, w_ref = refs[2 * nc_in:2 * nc_in + 3]
        out_refs = refs[2 * nc_in + 3:2 * nc_in + 3 + nc_out]
        (u_ref,) = refs[2 * nc_in + 3 + nc_out:]
        c = pl.program_id(1)
        dinv = dinv_ref[:, 0:1]

        @pl.when(c == 0)
        def _():
            parts = [s[j][...] + p[j][...] for j in range(nc_in)]
            sp = jnp.concatenate(parts, axis=1)
            u_ref[...] = jnp.maximum(dinv * sp + b_ref[...], 0.0)

        h = jnp.dot(u_ref[...], w_ref[...], preferred_element_type=F32,
                    precision=HI)
        pch = dinv * h
        for j, pr in enumerate(out_refs):
            @pl.when(c == j)
            def _(pr=pr):
                pr[...] = pch

    def call(s_parts, p_parts, dinv, b_row, w):
        grid = (NRB, nc_out)
        in_specs = (
            [pl.BlockSpec((RB, 128), lambda r, c: (r, 0))] * nc_in
            + [pl.BlockSpec((RB, 128), lambda r, c: (r, 0))] * nc_in
            + [
                pl.BlockSpec((RB, 128), lambda r, c: (r, 0)),
                pl.BlockSpec((1, d_in), lambda r, c: (0, 0)),
                pl.BlockSpec((d_in, 128), lambda r, c: (0, c)),
            ]
        )
        return pl.pallas_call(
            body,
            grid=grid,
            in_specs=in_specs,
            out_specs=[pl.BlockSpec((RB, 128), lambda r, c: (r, 0))] * nc_out,
            out_shape=[jax.ShapeDtypeStruct((NPAD, 128), F32)] * nc_out,
            scratch_shapes=[pltpu.VMEM((RB, d_in), F32)],
        )(*s_parts, *p_parts, dinv, b_row, w)

    return call


def _t4_body(s0_ref, s1_ref, p0_ref, p1_ref, dinv_ref, b_ref, bat_ref,
             lw1_ref, lb1_ref, lw2_ref, lb2_ref, lw3_ref, lb3_ref, out_ref):
    dinv = dinv_ref[:, 0:1]
    u0 = s0_ref[...] + p0_ref[...]
    u1 = s1_ref[...] + p1_ref[...]
    u = jnp.concatenate([u0, u1], axis=1)
    u = jnp.maximum(dinv * u + b_ref[...], 0.0)
    bat = bat_ref[...]
    onehot = (bat == lax.broadcasted_iota(jnp.int32, (G, NPAD), 0)).astype(F32)
    gsum = jnp.dot(onehot, u, preferred_element_type=F32, precision=HI)
    counts = jnp.sum(onehot, axis=1, keepdims=True)
    g = gsum / jnp.maximum(counts, 1.0)
    g = jnp.maximum(jnp.dot(g, lw1_ref[...], preferred_element_type=F32,
                            precision=HI) + lb1_ref[...], 0.0)
    g = jnp.maximum(jnp.dot(g, lw2_ref[...], preferred_element_type=F32,
                            precision=HI) + lb2_ref[...], 0.0)
    g = jnp.maximum(jnp.dot(g, lw3_ref[...], preferred_element_type=F32,
                            precision=HI) + lb3_ref[...], 0.0)
    out_ref[...] = g


def _t4(s_parts, p_parts, dinv, b_row, bat_row, lw1, lb1, lw2, lb2, lw3, lb3):
    return pl.pallas_call(
        _t4_body,
        out_shape=jax.ShapeDtypeStruct((G, 8), F32),
    )(*s_parts, *p_parts, dinv, b_row, bat_row,
      lw1, lb1, lw2, lb2, lw3, lb3)


_sc_scatter = functools.lru_cache(maxsize=None)(_make_sc_scatter)
_sc_deg = functools.lru_cache(maxsize=None)(_make_sc_deg)
_t2 = _make_t_mid(512, 4)
_t3 = _make_t_mid(512, 2)


def kernel(x, edge_index, batch, W1, b1, W2, b2, W3, b3,
           LW1, Lb1, LW2, Lb2, LW3, Lb3):
    src = jnp.concatenate(
        [edge_index[0].astype(jnp.int32),
         jnp.zeros((EPAD - E,), jnp.int32)]).reshape(EPAD // EB, EB)
    dst = jnp.concatenate(
        [edge_index[1].astype(jnp.int32),
         jnp.full((EPAD - E,), N, jnp.int32)]).reshape(EPAD // EB, EB)
    bat_row = jnp.concatenate(
        [batch.astype(jnp.int32),
         jnp.full((NPAD - N,), G, jnp.int32)]).reshape(1, NPAD)
    xp = jnp.pad(x, ((0, NPAD - N), (0, 0)))
    zeros128 = jnp.zeros((STRIPE, 128), F32)
    ones128 = jnp.ones((EB, 128), F32)

    dega, degb = _sc_deg()(dst, ones128, zeros128)
    dinv, *p1 = _t1(xp, W1, dega, degb)
    s1 = _sc_scatter(4)(src, dst, zeros128, *p1)
    p2 = _t2(s1, p1, dinv, b1.reshape(1, -1), W2)
    s2 = _sc_scatter(4)(src, dst, zeros128, *p2)
    p3 = _t3(s2, p2, dinv, b2.reshape(1, -1), W3)
    s3 = _sc_scatter(2)(src, dst, zeros128, *p3)
    return _t4(s3, p3, dinv, b3.reshape(1, -1), bat_row,
               LW1, Lb1.reshape(1, -1), LW2, Lb2.reshape(1, -1),
               LW3, Lb3.reshape(1, -1))


# speedup vs baseline: 6.0748x; 1.0263x over previous
import functools

import jax
import jax.numpy as jnp
from jax import lax
from jax.experimental import pallas as pl
from jax.experimental.pallas import tpu as pltpu
from jax.experimental.pallas import tpu_sc as plsc

N = 10000
E = 160000
G = 16
NPAD = 10240
EPAD = 163840
EB = 128
NB2 = EPAD // 16 // EB
NBH = NB2 // 2
STRIPE = NPAD // 16
RB = 2048
NRB = NPAD // RB
F32 = jnp.float32
HI = jax.lax.Precision.HIGHEST
_DN = (((1,), (0,)), ((), ()))


def _dot_x3(a, b):
    ah = a.astype(jnp.bfloat16)
    al = (a - ah.astype(F32)).astype(jnp.bfloat16)
    bh = b.astype(jnp.bfloat16)
    bl = (b - bh.astype(F32)).astype(jnp.bfloat16)

    def d(x, y):
        return lax.dot_general(x, y, _DN, preferred_element_type=F32)

    return d(ah, bl) + d(al, bh) + d(ah, bh)


def _sc_mesh():
    return plsc.VectorSubcoreMesh(core_axis_name="c", subcore_axis_name="s")


def _make_sc_deg():
    scratch = [
        pltpu.VMEM((NBH, EB), jnp.int32),
        pltpu.VMEM((EB, 128), F32),
        pltpu.VMEM_SHARED((NPAD, 128), F32),
    ]
    out_t = [jax.ShapeDtypeStruct((NPAD, 128), F32)] * 2

    @functools.partial(pl.kernel, mesh=_sc_mesh(), out_type=out_t,
                       scratch_types=scratch)
    def k(dst_hbm, ones_hbm, zero_hbm, d0_hbm, d1_hbm, dst_all, ones_v, acc):
        core = lax.axis_index("c")
        sub = lax.axis_index("s")
        wid = sub * 2 + core
        b0 = wid * NBH
        stripe0 = sub * STRIPE
        pltpu.sync_copy(ones_hbm, ones_v)
        pltpu.sync_copy(dst_hbm.at[pl.ds(b0, NBH), :], dst_all)
        pltpu.sync_copy(zero_hbm, acc.at[pl.ds(stripe0, STRIPE), :])
        plsc.subcore_barrier()

        def step(i, carry):
            pltpu.sync_copy(ones_v, acc.at[dst_all.at[i]], add=True)
            return carry

        lax.fori_loop(0, NBH, step, 0)
        plsc.subcore_barrier()

        @pl.when(core == 0)
        def _():
            pltpu.sync_copy(acc.at[pl.ds(stripe0, STRIPE), :],
                            d0_hbm.at[pl.ds(stripe0, STRIPE), :])

        @pl.when(core == 1)
        def _():
            pltpu.sync_copy(acc.at[pl.ds(stripe0, STRIPE), :],
                            d1_hbm.at[pl.ds(stripe0, STRIPE), :])

    return k


def _make_sc_scatter(n_chunks):
    cpc = (n_chunks + 1) // 2
    scratch = [
        pltpu.VMEM((NBH, EB), jnp.int32),
        pltpu.VMEM((NBH, EB), jnp.int32),
        pltpu.VMEM((2, EB, 128), F32),
        pltpu.VMEM_SHARED((NPAD, 128), F32),
        pltpu.SemaphoreType.DMA((2,)),
    ]
    out_t = [jax.ShapeDtypeStruct((NPAD, 128), F32)] * n_chunks

    @functools.partial(pl.kernel, mesh=_sc_mesh(), out_type=out_t,
                       scratch_types=scratch)
    def k(src_hbm, dst_hbm, zero_hbm, *rest):
        p_refs = rest[:n_chunks]
        s_refs = rest[n_chunks:2 * n_chunks]
        src_all, dst_all, rows, acc, sem = rest[2 * n_chunks:]
        core = lax.axis_index("c")
        sub = lax.axis_index("s")
        b0 = sub * NB2
        stripe0 = sub * STRIPE

        def process(p_ref, s_ref):
            pltpu.sync_copy(zero_hbm, acc.at[pl.ds(stripe0, STRIPE), :])
            plsc.subcore_barrier()

            for h in range(2):
                pltpu.sync_copy(src_hbm.at[pl.ds(b0 + h * NBH, NBH), :],
                                src_all)
                pltpu.sync_copy(dst_hbm.at[pl.ds(b0 + h * NBH, NBH), :],
                                dst_all)

                pltpu.async_copy(p_ref.at[src_all.at[0]], rows.at[0],
                                 sem.at[0])

                def step(i, carry):
                    par = lax.rem(i, 2)
                    nxt = lax.rem(i + 1, 2)

                    @pl.when(i + 1 < NBH)
                    def _():
                        pltpu.async_copy(p_ref.at[src_all.at[i + 1]],
                                         rows.at[nxt], sem.at[nxt])

                    pltpu.make_async_copy(p_ref.at[src_all.at[i]],
                                          rows.at[par], sem.at[par]).wait()
                    pltpu.sync_copy(rows.at[par], acc.at[dst_all.at[i]],
                                    add=True)
                    return carry

                lax.fori_loop(0, NBH, step, 0)
            plsc.subcore_barrier()
            pltpu.sync_copy(acc.at[pl.ds(stripe0, STRIPE), :],
                            s_ref.at[pl.ds(stripe0, STRIPE), :])

        for li in range(cpc):
            c0 = li
            c1 = cpc + li

            @pl.when(core == 0)
            def _(c0=c0):
                process(p_refs[c0], s_refs[c0])

            if c1 < n_chunks:
                @pl.when(core == 1)
                def _(c1=c1):
                    process(p_refs[c1], s_refs[c1])

    return k


def _t1_body(x_ref, w_ref, dega_ref, degb_ref, dinv_ref, *p_refs):
    c = pl.program_id(1)
    deg = dega_ref[:, 0:1] + degb_ref[:, 0:1] + 1.0
    dinv = jax.lax.rsqrt(deg)
    dinv_ref[...] = jnp.broadcast_to(dinv, (RB, 128))
    h = _dot_x3(x_ref[...], w_ref[...])
    pch = dinv * h
    for j, pr in enumerate(p_refs):
        @pl.when(c == j)
        def _(pr=pr):
            pr[...] = pch


def _t1(xp, w1, dega, degb):
    grid = (NRB, 4)
    return pl.pallas_call(
        _t1_body,
        grid=grid,
        in_specs=[
            pl.BlockSpec((RB, 256), lambda r, c: (r, 0)),
            pl.BlockSpec((256, 128), lambda r, c: (0, c)),
            pl.BlockSpec((RB, 128), lambda r, c: (r, 0)),
            pl.BlockSpec((RB, 128), lambda r, c: (r, 0)),
        ],
        out_specs=[pl.BlockSpec((RB, 128), lambda r, c: (r, 0))] * 5,
        out_shape=[jax.ShapeDtypeStruct((NPAD, 128), F32)] * 5,
    )(xp, w1, dega, degb)


def _make_t_mid(d_in, nc_out):
    nc_in = d_in // 128

    def body(*refs):
        s = refs[:nc_in]
        p = refs[nc_in:2 * nc_in]
        dinv_ref, b_ref, w_ref = refs[2 * nc_in:2 * nc_in + 3]
        out_refs = refs[2 * nc_in + 3:2 * nc_in + 3 + nc_out]
        (u_ref,) = refs[2 * nc_in + 3 + nc_out:]
        c = pl.program_id(1)
        dinv = dinv_ref[:, 0:1]

        @pl.when(c == 0)
        def _():
            parts = [s[j][...] + p[j][...] for j in range(nc_in)]
            sp = jnp.concatenate(parts, axis=1)
            u_ref[...] = jnp.maximum(dinv * sp + b_ref[...], 0.0)

        h = _dot_x3(u_ref[...], w_ref[...])
        pch = dinv * h
        for j, pr in enumerate(out_refs):
            @pl.when(c == j)
            def _(pr=pr):
                pr[...] = pch

    def call(s_parts, p_parts, dinv, b_row, w):
        grid = (NRB, nc_out)
        in_specs = (
            [pl.BlockSpec((RB, 128), lambda r, c: (r, 0))] * nc_in
            + [pl.BlockSpec((RB, 128), lambda r, c: (r, 0))] * nc_in
            + [
                pl.BlockSpec((RB, 128), lambda r, c: (r, 0)),
                pl.BlockSpec((1, d_in), lambda r, c: (0, 0)),
                pl.BlockSpec((d_in, 128), lambda r, c: (0, c)),
            ]
        )
        return pl.pallas_call(
            body,
            grid=grid,
            in_specs=in_specs,
            out_specs=[pl.BlockSpec((RB, 128), lambda r, c: (r, 0))] * nc_out,
            out_shape=[jax.ShapeDtypeStruct((NPAD, 128), F32)] * nc_out,
            scratch_shapes=[pltpu.VMEM((RB, d_in), F32)],
        )(*s_parts, *p_parts, dinv, b_row, w)

    return call


def _t4_body(s0_ref, s1_ref, p0_ref, p1_ref, dinv_ref, b_ref, bat_ref,
             lw1_ref, lb1_ref, lw2_ref, lb2_ref, lw3_ref, lb3_ref, out_ref):
    dinv = dinv_ref[:, 0:1]
    u0 = s0_ref[...] + p0_ref[...]
    u1 = s1_ref[...] + p1_ref[...]
    u = jnp.concatenate([u0, u1], axis=1)
    u = jnp.maximum(dinv * u + b_ref[...], 0.0)
    bat = bat_ref[...]
    onehot = (bat == lax.broadcasted_iota(jnp.int32, (G, NPAD), 0)).astype(F32)
    gsum = jnp.dot(onehot, u, preferred_element_type=F32, precision=HI)
    counts = jnp.sum(onehot, axis=1, keepdims=True)
    g = gsum / jnp.maximum(counts, 1.0)
    g = jnp.maximum(_dot_x3(g, lw1_ref[...]) + lb1_ref[...], 0.0)
    g = jnp.maximum(_dot_x3(g, lw2_ref[...]) + lb2_ref[...], 0.0)
    g = jnp.maximum(_dot_x3(g, lw3_ref[...]) + lb3_ref[...], 0.0)
    out_ref[...] = g


def _t4(s_parts, p_parts, dinv, b_row, bat_row, lw1, lb1, lw2, lb2, lw3, lb3):
    return pl.pallas_call(
        _t4_body,
        out_shape=jax.ShapeDtypeStruct((G, 8), F32),
    )(*s_parts, *p_parts, dinv, b_row, bat_row,
      lw1, lb1, lw2, lb2, lw3, lb3)


_sc_scatter = functools.lru_cache(maxsize=None)(_make_sc_scatter)
_sc_deg = functools.lru_cache(maxsize=None)(_make_sc_deg)
_t2 = _make_t_mid(512, 4)
_t3 = _make_t_mid(512, 2)


def kernel(x, edge_index, batch, W1, b1, W2, b2, W3, b3,
           LW1, Lb1, LW2, Lb2, LW3, Lb3):
    src = jnp.concatenate(
        [edge_index[0].astype(jnp.int32),
         jnp.zeros((EPAD - E,), jnp.int32)]).reshape(EPAD // EB, EB)
    dst = jnp.concatenate(
        [edge_index[1].astype(jnp.int32),
         jnp.full((EPAD - E,), N, jnp.int32)]).reshape(EPAD // EB, EB)
    bat_row = jnp.concatenate(
        [batch.astype(jnp.int32),
         jnp.full((NPAD - N,), G, jnp.int32)]).reshape(1, NPAD)
    xp = jnp.pad(x, ((0, NPAD - N), (0, 0)))
    zeros128 = jnp.zeros((STRIPE, 128), F32)
    ones128 = jnp.ones((EB, 128), F32)

    dega, degb = _sc_deg()(dst, ones128, zeros128)
    dinv, *p1 = _t1(xp, W1, dega, degb)
    s1 = _sc_scatter(4)(src, dst, zeros128, *p1)
    p2 = _t2(s1, p1, dinv, b1.reshape(1, -1), W2)
    s2 = _sc_scatter(4)(src, dst, zeros128, *p2)
    p3 = _t3(s2, p2, dinv, b2.reshape(1, -1), W3)
    s3 = _sc_scatter(2)(src, dst, zeros128, *p3)
    return _t4(s3, p3, dinv, b3.reshape(1, -1), bat_row,
               LW1, Lb1.reshape(1, -1), LW2, Lb2.reshape(1, -1),
               LW3, Lb3.reshape(1, -1))
